# Initial kernel scaffold; baseline (speedup 1.0000x reference)
import jax
import jax.numpy as jnp
from jax.experimental import pallas as pl


def kernel(x, W1, g1, b1, W2, g2, b2, Wn, gn, bn, Wout, bout):
    raise NotImplementedError("write your pallas kernel here")


# speedup vs baseline: 19.6452x; 19.6452x over previous
import jax
import jax.numpy as jnp
from jax.experimental import pallas as pl
from jax.experimental.pallas import tpu as pltpu

_B, _C, _N, _H, _K = 16, 7, 1024, 64, 16
_BIG = 3.0e38


def _tc_body(x_ref, w1_ref, b1_ref, w2_ref, b2_ref, wa_ref, bn_ref,
             wc_ref, wo_ref, bo_ref, out_ref):
    f32 = jnp.float32
    xb = x_ref[0]
    h1 = jnp.maximum(
        jax.lax.dot_general(w1_ref[...], xb, (((1,), (0,)), ((), ())),
                            preferred_element_type=f32) + b1_ref[...], 0.0)
    h2 = jnp.maximum(
        jax.lax.dot_general(w2_ref[...], h1, (((1,), (0,)), ((), ())),
                            preferred_element_type=f32) + b2_ref[...], 0.0)
    sq = jnp.sum(h2 * h2, axis=0)
    gram = jax.lax.dot_general(h2, h2, (((0,), (0,)), ((), ())),
                               preferred_element_type=f32)
    dist = (sq[:, None] - 2.0 * gram) + sq[None, :]
    a_rows = jax.lax.dot_general(h2, wa_ref[...], (((0,), (1,)), ((), ())),
                                 preferred_element_type=f32) + bn_ref[...]
    c_rows = jax.lax.dot_general(h2, wc_ref[...], (((0,), (1,)), ((), ())),
                                 preferred_element_type=f32)
    jcol = jax.lax.broadcasted_iota(jnp.int32, (_N, _N), 1)
    cmax = jnp.full((_N, _H), -_BIG, dtype=f32)
    for _ in range(_K):
        m = jnp.min(dist, axis=1, keepdims=True)
        amin = jnp.min(jnp.where(dist == m, jcol, _N), axis=1,
                       keepdims=True)
        onehot = jcol == amin
        g = jax.lax.dot_general(onehot.astype(f32), c_rows,
                                (((1,), (0,)), ((), ())),
                                preferred_element_type=f32)
        cmax = jnp.maximum(cmax, g)
        dist = jnp.where(onehot, _BIG, dist)
    u = jnp.maximum(a_rows + cmax, 0.0)
    s = jax.lax.dot_general(u, wo_ref[...], (((1,), (0,)), ((), ())),
                            preferred_element_type=f32) + bo_ref[...]
    out_ref[0, 0, :] = s[:, 0]


def _scores_pallas(x, w1f, b1c, w2f, b2c, wa, bnr, wc, woc, boc,
                   interpret=False):
    full = lambda shape: pl.BlockSpec(shape, lambda b: (0,) * len(shape))
    out = pl.pallas_call(
        _tc_body,
        grid=(_B,),
        in_specs=[
            pl.BlockSpec((1, _C, _N), lambda b: (b, 0, 0)),
            full((_H, _C)), full((_H, 1)), full((_H, _H)), full((_H, 1)),
            full((_H, _H)), full((1, _H)), full((_H, _H)), full((_H, 1)),
            full((1, 1)),
        ],
        out_specs=pl.BlockSpec((1, 1, _N), lambda b: (b, 0, 0)),
        out_shape=jax.ShapeDtypeStruct((_B, 1, _N), jnp.float32),
        compiler_params=pltpu.CompilerParams(
            dimension_semantics=("arbitrary",)),
        interpret=interpret,
    )(x, w1f, b1c, w2f, b2c, wa, bnr, wc, woc, boc)
    return out.reshape(_B, _N)


def kernel(x, W1, g1, b1, W2, g2, b2, Wn, gn, bn, Wout, bout):
    w1f = g1[:, None] * W1
    w2f = g2[:, None] * W2
    wc_part = Wn[:, :_H]
    wd_part = Wn[:, _H:]
    wa = gn[:, None] * (wc_part - wd_part)
    wc = gn[:, None] * wd_part
    return _scores_pallas(
        x, w1f, b1[:, None], w2f, b2[:, None], wa, bn[None, :], wc,
        Wout.reshape(_H, 1), bout.reshape(1, 1))


# speedup vs baseline: 21.4294x; 1.0908x over previous
import jax
import jax.numpy as jnp
from jax.experimental import pallas as pl
from jax.experimental.pallas import tpu as pltpu

_B, _C, _N, _H, _K = 16, 7, 1024, 64, 16
_BIG = 3.0e38


def _tc_body(x_ref, w1_ref, b1_ref, w2_ref, b2_ref, wa_ref, bn_ref,
             wc_ref, wo_ref, bo_ref, out_ref):
    f32 = jnp.float32
    bf16 = jnp.bfloat16
    xb = x_ref[0]
    h1 = jnp.maximum(
        jax.lax.dot_general(w1_ref[...].astype(bf16), xb.astype(bf16),
                            (((1,), (0,)), ((), ())),
                            preferred_element_type=f32) + b1_ref[...], 0.0)
    h2 = jnp.maximum(
        jax.lax.dot_general(w2_ref[...].astype(bf16), h1.astype(bf16),
                            (((1,), (0,)), ((), ())),
                            preferred_element_type=f32) + b2_ref[...], 0.0)
    h2sq = h2 * h2
    sq_row = jnp.sum(h2sq, axis=0, keepdims=True)
    sq_col = jax.lax.dot_general(
        h2sq, jnp.ones((_H, 1), f32), (((0,), (0,)), ((), ())),
        preferred_element_type=f32)
    h2b = h2.astype(bf16)
    gram = jax.lax.dot_general(h2b, h2b, (((0,), (0,)), ((), ())),
                               preferred_element_type=f32)
    dist = (sq_row - 2.0 * gram) + sq_col
    a_t = jax.lax.dot_general(wa_ref[...], h2, (((1,), (0,)), ((), ())),
                              preferred_element_type=f32) + bn_ref[...]
    c_t = jax.lax.dot_general(wc_ref[...], h2, (((1,), (0,)), ((), ())),
                              preferred_element_type=f32)
    irow = jax.lax.broadcasted_iota(jnp.int32, (_N, _N), 0)
    cmax = jnp.full((_H, _N), -_BIG, dtype=f32)
    for _ in range(_K):
        m = jnp.min(dist, axis=0, keepdims=True)
        amin = jnp.min(jnp.where(dist == m, irow, _N), axis=0,
                       keepdims=True)
        onehot = irow == amin
        g = jax.lax.dot_general(c_t, onehot.astype(f32),
                                (((1,), (0,)), ((), ())),
                                preferred_element_type=f32)
        cmax = jnp.maximum(cmax, g)
        dist = jnp.where(onehot, _BIG, dist)
    u = jnp.maximum(a_t + cmax, 0.0)
    s = jax.lax.dot_general(wo_ref[...], u, (((1,), (0,)), ((), ())),
                            preferred_element_type=f32) + bo_ref[...]
    out_ref[0, 0, :] = s[0, :]


def _scores_pallas(x, w1f, b1c, w2f, b2c, wa, bnc, wc, wor, boc,
                   interpret=False):
    full = lambda shape: pl.BlockSpec(shape, lambda b: (0,) * len(shape))
    out = pl.pallas_call(
        _tc_body,
        grid=(_B,),
        in_specs=[
            pl.BlockSpec((1, _C, _N), lambda b: (b, 0, 0)),
            full((_H, _C)), full((_H, 1)), full((_H, _H)), full((_H, 1)),
            full((_H, _H)), full((_H, 1)), full((_H, _H)), full((1, _H)),
            full((1, 1)),
        ],
        out_specs=pl.BlockSpec((1, 1, _N), lambda b: (b, 0, 0)),
        out_shape=jax.ShapeDtypeStruct((_B, 1, _N), jnp.float32),
        compiler_params=pltpu.CompilerParams(
            dimension_semantics=("arbitrary",)),
        interpret=interpret,
    )(x, w1f, b1c, w2f, b2c, wa, bnc, wc, wor, boc)
    return out.reshape(_B, _N)


def kernel(x, W1, g1, b1, W2, g2, b2, Wn, gn, bn, Wout, bout):
    w1f = g1[:, None] * W1
    w2f = g2[:, None] * W2
    wc_part = Wn[:, :_H]
    wd_part = Wn[:, _H:]
    wa = gn[:, None] * (wc_part - wd_part)
    wc = gn[:, None] * wd_part
    return _scores_pallas(
        x, w1f, b1[:, None], w2f, b2[:, None], wa, bn[:, None], wc,
        Wout.reshape(1, _H), bout.reshape(1, 1))
